# baseline (speedup 1.0000x reference)
import functools

import jax
import jax.numpy as jnp
from jax import lax
from jax.experimental import pallas as pl
from jax.experimental.pallas import tpu as pltpu
from jax.experimental.pallas import tpu_sc as plsc

NUM_VEC = 8192
DIM = 64
NC = 2
NS = 16
NW = NC * NS
ROWS_PER_W = NUM_VEC // NW
CHUNK = ROWS_PER_W // 2


@functools.partial(
    pl.kernel,
    mesh=plsc.VectorSubcoreMesh(core_axis_name="c", subcore_axis_name="s"),
    out_type=jax.ShapeDtypeStruct((NUM_VEC, DIM), jnp.float32),
    scratch_types=[
        pltpu.VMEM((CHUNK, DIM), jnp.float32),
        pltpu.VMEM((CHUNK, DIM), jnp.float32),
        pltpu.SemaphoreType.DMA,
        pltpu.SemaphoreType.DMA,
        pltpu.SemaphoreType.DMA,
        pltpu.SemaphoreType.DMA,
    ],
)
def _sc_copy(emb_hbm, out_hbm, buf0, buf1, in0, in1, out0, out1):
    wid = lax.axis_index("s") * NC + lax.axis_index("c")
    base = wid * ROWS_PER_W
    r0 = pltpu.make_async_copy(emb_hbm.at[pl.ds(base, CHUNK)], buf0, in0)
    r1 = pltpu.make_async_copy(emb_hbm.at[pl.ds(base + CHUNK, CHUNK)], buf1, in1)
    r0.start()
    r1.start()
    r0.wait()
    w0 = pltpu.make_async_copy(buf0, out_hbm.at[pl.ds(base, CHUNK)], out0)
    w0.start()
    r1.wait()
    w1 = pltpu.make_async_copy(buf1, out_hbm.at[pl.ds(base + CHUNK, CHUNK)], out1)
    w1.start()
    w0.wait()
    w1.wait()


def kernel(embeddings):
    return _sc_copy(embeddings)
